# baseline (speedup 1.0000x reference)
import functools

import jax
import jax.numpy as jnp
from jax import lax
from jax.experimental import pallas as pl
from jax.experimental.pallas import tpu as pltpu
from jax.experimental.pallas import tpu_sc as plsc

VOCAB = 1000000
D = 64
SEQ = 200
NTOK = 4096

_info = plsc.get_sparse_core_info()
NC, NS = _info.num_cores, _info.num_subcores
NW = NC * NS
N_PAIRS = SEQ * (NTOK // 256)
P_PER_W = N_PAIRS // NW


def _emb_kernel(
    table_hbm, xt_hbm, out_hbm,
    idx0_v, idx1_v, rows0_v, rows1_v, slab0_v, slab1_v, gsem, wsem,
):
    wid = lax.axis_index("s") * NC + lax.axis_index("c")
    p0 = wid * P_PER_W
    idx_v = (idx0_v, idx1_v)
    rows_v = (rows0_v, rows1_v)
    slab_v = (slab0_v, slab1_v)

    def start_gather(p, buf):
        i1 = p // 16
        bp = p % 16
        pltpu.sync_copy(xt_hbm.at[i1, pl.ds(bp * 256, 256)], idx_v[buf])
        pltpu.async_copy(table_hbm.at[idx_v[buf]], rows_v[buf], gsem)

    def wait_gather(buf):
        pltpu.make_async_copy(table_hbm.at[idx_v[buf]], rows_v[buf], gsem).wait()

    def start_writes(p, buf):
        i1 = p // 16
        b0 = 2 * (p % 16)
        for d in range(2):
            pltpu.async_copy(
                slab_v[buf].at[d], out_hbm.at[i1, pl.ds(0, 8), b0 + d], wsem
            )

    def wait_writes(buf):
        for d in range(2):
            pltpu.make_async_copy(
                slab_v[buf].at[d], out_hbm.at[0, pl.ds(0, 8), 0], wsem
            ).wait()

    def transpose_block(buf):
        rows = rows_v[buf]
        slab = slab_v[buf]
        base = lax.iota(jnp.int32, 16)
        for d in range(2):
            for kg in range(8):
                row_ids = base + (128 * d + 16 * kg)
                for f in range(64):
                    col_ids = jnp.full((16,), f, jnp.int32)
                    v = plsc.load_gather(rows, [row_ids, col_ids])
                    slab[d, f // 8, f % 8, pl.ds(16 * kg, 16)] = v

    start_gather(p0, 0)

    def body(g, carry):
        for buf in range(2):
            i = 2 * g + buf
            nbuf = 1 - buf

            @pl.when(i + 1 < P_PER_W)
            def _():
                start_gather(p0 + i + 1, nbuf)

            wait_gather(buf)

            @pl.when(i >= 2)
            def _():
                wait_writes(buf)

            transpose_block(buf)
            start_writes(p0 + i, buf)
        return carry

    lax.fori_loop(0, P_PER_W // 2, body, 0)

    for buf in range(2):
        wait_writes(buf)


@jax.jit
def _embed(xt, table):
    mesh = plsc.VectorSubcoreMesh(core_axis_name="c", subcore_axis_name="s")
    f = functools.partial(
        pl.kernel,
        mesh=mesh,
        out_type=jax.ShapeDtypeStruct((SEQ, 8, NTOK // 128, 8, 128), jnp.float32),
        scratch_types=[
            pltpu.VMEM((256,), jnp.int32),
            pltpu.VMEM((256,), jnp.int32),
            pltpu.VMEM((256, D), jnp.float32),
            pltpu.VMEM((256, D), jnp.float32),
            pltpu.VMEM((2, 8, 8, 128), jnp.float32),
            pltpu.VMEM((2, 8, 8, 128), jnp.float32),
            pltpu.SemaphoreType.DMA,
            pltpu.SemaphoreType.DMA,
        ],
        compiler_params=pltpu.CompilerParams(
            needs_layout_passes=False, use_tc_tiling_on_sc=False
        ),
    )(_emb_kernel)
    return f(table, xt)


def kernel(x, table):
    xt = x.T
    out5 = _embed(xt, table)
    return out5.transpose(2, 4, 0, 1, 3).reshape(NTOK, SEQ, D)


# speedup vs baseline: 1.3589x; 1.3589x over previous
import functools

import jax
import jax.numpy as jnp
from jax import lax
from jax.experimental import pallas as pl
from jax.experimental.pallas import tpu as pltpu
from jax.experimental.pallas import tpu_sc as plsc

VOCAB = 1000000
D = 64
SEQ = 200
NTOK = 4096
B2 = 2 * NTOK * SEQ

_info = plsc.get_sparse_core_info()
NC, NS = _info.num_cores, _info.num_subcores
NW = NC * NS
B_PER_W = B2 // NW
CHUNK = 800
N_CHUNKS = B_PER_W // CHUNK
NBUF = 2


def _emb_kernel(table_hbm, idx_hbm, out_hbm, idx_v, rows_v, gsem, wsem):
    wid = lax.axis_index("s") * NC + lax.axis_index("c")
    base = wid * B_PER_W

    def start_gather(i, b):
        off = base + i * CHUNK
        pltpu.sync_copy(idx_hbm.at[pl.ds(off, CHUNK)], idx_v.at[b])
        pltpu.async_copy(table_hbm.at[idx_v.at[b]], rows_v.at[b], gsem)

    def wait_gather(b):
        pltpu.make_async_copy(table_hbm.at[idx_v.at[b]], rows_v.at[b], gsem).wait()

    def start_write(i, b):
        off = base + i * CHUNK
        pltpu.async_copy(rows_v.at[b], out_hbm.at[pl.ds(off, CHUNK)], wsem)

    def wait_write(b):
        pltpu.make_async_copy(
            rows_v.at[b], out_hbm.at[pl.ds(base, CHUNK)], wsem
        ).wait()

    start_gather(0, 0)

    def body(g, carry):
        for b in range(NBUF):
            i = g * NBUF + b
            nb = (b + 1) % NBUF

            @pl.when(i >= 1)
            def _():
                wait_write(nb)

            @pl.when(i + 1 < N_CHUNKS)
            def _():
                start_gather(i + 1, nb)

            wait_gather(b)
            start_write(i, b)
        return carry

    lax.fori_loop(0, N_CHUNKS // NBUF, body, 0)

    wait_write((N_CHUNKS - 1) % NBUF)


@jax.jit
def _embed(xf2, table):
    mesh = plsc.VectorSubcoreMesh(core_axis_name="c", subcore_axis_name="s")
    f = functools.partial(
        pl.kernel,
        mesh=mesh,
        out_type=jax.ShapeDtypeStruct((B2, D), jnp.float32),
        scratch_types=[
            pltpu.VMEM((NBUF, CHUNK), jnp.int32),
            pltpu.VMEM((NBUF, CHUNK, D), jnp.float32),
            pltpu.SemaphoreType.DMA,
            pltpu.SemaphoreType.DMA,
        ],
        compiler_params=pltpu.CompilerParams(use_tc_tiling_on_sc=False),
    )(_emb_kernel)
    return f(table, xf2)


def kernel(x, table):
    xf2 = jnp.repeat(x.reshape(-1), 2)
    out2 = _embed(xf2, table)
    return out2.reshape(NTOK, SEQ, 2 * D)[..., :D]


# speedup vs baseline: 2.2301x; 1.6411x over previous
import functools

import jax
import jax.numpy as jnp
from jax import lax
from jax.experimental import pallas as pl
from jax.experimental.pallas import tpu as pltpu
from jax.experimental.pallas import tpu_sc as plsc

VOCAB = 1000000
D = 64
SEQ = 200
NTOK = 4096
B = NTOK * SEQ

_info = plsc.get_sparse_core_info()
NC, NS = _info.num_cores, _info.num_subcores
NW = NC * NS
B_PER_W = B // NW
CHUNK = 800
N_CHUNKS = B_PER_W // CHUNK
NBUF = 2


def _emb_kernel(table_hbm, idx_hbm, out_hbm, idx_v, rows_v, gsem, wsem):
    wid = lax.axis_index("s") * NC + lax.axis_index("c")
    base = wid * B_PER_W

    def start_gather(i, b):
        off = base + i * CHUNK
        pltpu.sync_copy(idx_hbm.at[pl.ds(off, CHUNK)], idx_v.at[b])
        pltpu.async_copy(table_hbm.at[idx_v.at[b]], rows_v.at[b], gsem)

    def wait_gather(b):
        pltpu.make_async_copy(table_hbm.at[idx_v.at[b]], rows_v.at[b], gsem).wait()

    def start_write(i, b):
        off = base + i * CHUNK
        pltpu.async_copy(
            rows_v.at[b], out_hbm.at[pl.ds(off, CHUNK), pl.ds(0, D)], wsem
        )

    def wait_write(b):
        pltpu.make_async_copy(
            rows_v.at[b], out_hbm.at[pl.ds(base, CHUNK), pl.ds(0, D)], wsem
        ).wait()

    start_gather(0, 0)

    def body(g, carry):
        for b in range(NBUF):
            i = g * NBUF + b
            nb = (b + 1) % NBUF

            @pl.when(i >= 1)
            def _():
                wait_write(nb)

            @pl.when(i + 1 < N_CHUNKS)
            def _():
                start_gather(i + 1, nb)

            wait_gather(b)
            start_write(i, b)
        return carry

    lax.fori_loop(0, N_CHUNKS // NBUF, body, 0)

    wait_write((N_CHUNKS - 1) % NBUF)


@jax.jit
def _embed(xf, table):
    mesh = plsc.VectorSubcoreMesh(core_axis_name="c", subcore_axis_name="s")
    f = functools.partial(
        pl.kernel,
        mesh=mesh,
        out_type=jax.ShapeDtypeStruct((B, 2 * D), jnp.float32),
        scratch_types=[
            pltpu.VMEM((NBUF, CHUNK), jnp.int32),
            pltpu.VMEM((NBUF, CHUNK, D), jnp.float32),
            pltpu.SemaphoreType.DMA,
            pltpu.SemaphoreType.DMA,
        ],
        compiler_params=pltpu.CompilerParams(use_tc_tiling_on_sc=False),
    )(_emb_kernel)
    return f(table, xf)


def kernel(x, table):
    xf = x.reshape(-1)
    out2 = _embed(xf, table)
    return out2.reshape(NTOK, SEQ, 2 * D)[..., :D]


# speedup vs baseline: 2.3893x; 1.0714x over previous
import functools

import jax
import jax.numpy as jnp
from jax import lax
from jax.experimental import pallas as pl
from jax.experimental.pallas import tpu as pltpu
from jax.experimental.pallas import tpu_sc as plsc

VOCAB = 1000000
D = 64
SEQ = 200
NTOK = 4096
B = NTOK * SEQ

_info = plsc.get_sparse_core_info()
NC, NS = _info.num_cores, _info.num_subcores
NW = NC * NS
B_PER_W = B // NW
CHUNK = 800
N_CHUNKS = B_PER_W // CHUNK
NBUF = 2


def _emb_kernel(table_hbm, idx_hbm, out_hbm, idx_v, rows_v, gsem, wsem):
    wid = lax.axis_index("s") * NC + lax.axis_index("c")
    base = wid * B_PER_W

    def start_gather(i, b):
        off = base + i * CHUNK
        pltpu.sync_copy(idx_hbm.at[pl.ds(off, CHUNK)], idx_v.at[b])
        pltpu.async_copy(table_hbm.at[idx_v.at[b]], rows_v.at[b], gsem)

    def wait_gather(b):
        pltpu.make_async_copy(table_hbm.at[idx_v.at[b]], rows_v.at[b], gsem).wait()

    def start_write(i, b):
        off = base + i * CHUNK
        pltpu.async_copy(
            rows_v.at[b], out_hbm.at[pl.ds(off, CHUNK), pl.ds(0, D)], wsem
        )

    def wait_write(b):
        pltpu.make_async_copy(
            rows_v.at[b], out_hbm.at[pl.ds(base, CHUNK), pl.ds(0, D)], wsem
        ).wait()

    start_gather(0, 0)

    def body(g, carry):
        for b in range(NBUF):
            i = g * NBUF + b
            nb = (b + 1) % NBUF

            @pl.when(i >= 1)
            def _():
                wait_write(nb)

            @pl.when(i + 1 < N_CHUNKS)
            def _():
                start_gather(i + 1, nb)

            wait_gather(b)
            start_write(i, b)
        return carry

    lax.fori_loop(0, N_CHUNKS // NBUF, body, 0)

    wait_write((N_CHUNKS - 1) % NBUF)


@jax.jit
def _embed(xf, table):
    mesh = plsc.VectorSubcoreMesh(core_axis_name="c", subcore_axis_name="s")
    f = functools.partial(
        pl.kernel,
        mesh=mesh,
        out_type=jax.ShapeDtypeStruct((B, 2 * D), jnp.float32),
        scratch_types=[
            pltpu.VMEM((NBUF, CHUNK), jnp.int32),
            pltpu.VMEM((NBUF, CHUNK, D), jnp.float32),
            pltpu.SemaphoreType.DMA,
            pltpu.SemaphoreType.DMA,
        ],
        compiler_params=pltpu.CompilerParams(use_tc_tiling_on_sc=False),
    )(_emb_kernel)
    return f(table, xf)


def kernel(x, table):
    tpad = jnp.pad(table, ((0, 0), (0, D)))
    t2 = tpad.reshape(2 * VOCAB, D)
    xf = 2 * x.reshape(-1)
    out2 = _embed(xf, t2)
    return out2.reshape(NTOK, SEQ, 2 * D)[..., :D]
